# baseline (device time: 42506 ns/iter reference)
import jax
import jax.numpy as jnp
from jax import lax
from jax.experimental import pallas as pl
from jax.experimental.pallas import tpu as pltpu

N_DEV = 8


def kernel(A, B):
    m, k_per = A.shape
    _, n = B.shape
    m_blk = m // N_DEV

    def body(a_ref, b_ref, out_ref, send_buf, comm_buf, send_sems, recv_sems):
        my_pos = lax.axis_index("i")

        barrier_sem = pltpu.get_barrier_semaphore()
        for p in range(1, N_DEV):
            peer = (my_pos + p) % N_DEV
            pl.semaphore_signal(
                barrier_sem, inc=1,
                device_id=(peer,), device_id_type=pl.DeviceIdType.MESH,
            )
        pl.semaphore_wait(barrier_sem, N_DEV - 1)

        b = b_ref[...].astype(jnp.bfloat16)

        rdmas = []
        for k in range(1, N_DEV):
            target = (my_pos + k) % N_DEV
            a_chunk = a_ref[pl.ds(target * m_blk, m_blk), :].astype(jnp.bfloat16)
            send_buf[k, :, :] = jnp.dot(
                a_chunk, b, preferred_element_type=jnp.float32
            )
            rdma = pltpu.make_async_remote_copy(
                src_ref=send_buf.at[k],
                dst_ref=comm_buf.at[k],
                send_sem=send_sems.at[k],
                recv_sem=recv_sems.at[k],
                device_id=(target,),
                device_id_type=pl.DeviceIdType.MESH,
            )
            rdma.start()
            rdmas.append(rdma)

        a_own = a_ref[pl.ds(my_pos * m_blk, m_blk), :].astype(jnp.bfloat16)
        out_ref[...] = jnp.dot(a_own, b, preferred_element_type=jnp.float32)

        for k in range(1, N_DEV):
            rdmas[k - 1].wait_recv()
            out_ref[...] += comm_buf[k, :, :]

        for k in range(1, N_DEV):
            rdmas[k - 1].wait_send()

    return pl.pallas_call(
        body,
        out_shape=jax.ShapeDtypeStruct((m_blk, n), jnp.float32),
        in_specs=[
            pl.BlockSpec(memory_space=pltpu.VMEM),
            pl.BlockSpec(memory_space=pltpu.VMEM),
        ],
        out_specs=pl.BlockSpec(memory_space=pltpu.VMEM),
        scratch_shapes=[
            pltpu.VMEM((N_DEV, m_blk, n), jnp.float32),
            pltpu.VMEM((N_DEV, m_blk, n), jnp.float32),
            pltpu.SemaphoreType.DMA((N_DEV,)),
            pltpu.SemaphoreType.DMA((N_DEV,)),
        ],
        compiler_params=pltpu.CompilerParams(collective_id=0),
    )(A, B)


# device time: 25120 ns/iter; 1.6921x vs baseline; 1.6921x over previous
import jax
import jax.numpy as jnp
from jax import lax
from jax.experimental import pallas as pl
from jax.experimental.pallas import tpu as pltpu

N_DEV = 8


def kernel(A, B):
    m, k_per = A.shape
    _, n = B.shape
    m_blk = m // N_DEV

    def body(a_ref, b_ref, out_ref, send_buf, comm_buf, send_sems, recv_sems):
        my_pos = lax.axis_index("i")

        barrier_sem = pltpu.get_barrier_semaphore()
        for p in range(1, N_DEV):
            peer = (my_pos + p) % N_DEV
            pl.semaphore_signal(
                barrier_sem, inc=1,
                device_id=(peer,), device_id_type=pl.DeviceIdType.MESH,
            )
        pl.semaphore_wait(barrier_sem, N_DEV - 1)

        b = b_ref[...].astype(jnp.bfloat16)

        rdmas = []
        for k in range(1, N_DEV):
            target = (my_pos + k) % N_DEV
            a_chunk = a_ref[pl.ds(target * m_blk, m_blk), :].astype(jnp.bfloat16)
            send_buf[k, :, :] = jnp.dot(
                a_chunk, b, preferred_element_type=jnp.float32
            ).astype(jnp.bfloat16)
            rdma = pltpu.make_async_remote_copy(
                src_ref=send_buf.at[k],
                dst_ref=comm_buf.at[k],
                send_sem=send_sems.at[k],
                recv_sem=recv_sems.at[k],
                device_id=(target,),
                device_id_type=pl.DeviceIdType.MESH,
            )
            rdma.start()
            rdmas.append(rdma)

        a_own = a_ref[pl.ds(my_pos * m_blk, m_blk), :].astype(jnp.bfloat16)
        out_ref[...] = jnp.dot(a_own, b, preferred_element_type=jnp.float32)

        for k in range(1, N_DEV):
            rdmas[k - 1].wait_recv()
            out_ref[...] += comm_buf[k, :, :].astype(jnp.float32)

        for k in range(1, N_DEV):
            rdmas[k - 1].wait_send()

    return pl.pallas_call(
        body,
        out_shape=jax.ShapeDtypeStruct((m_blk, n), jnp.float32),
        in_specs=[
            pl.BlockSpec(memory_space=pltpu.VMEM),
            pl.BlockSpec(memory_space=pltpu.VMEM),
        ],
        out_specs=pl.BlockSpec(memory_space=pltpu.VMEM),
        scratch_shapes=[
            pltpu.VMEM((N_DEV, m_blk, n), jnp.bfloat16),
            pltpu.VMEM((N_DEV, m_blk, n), jnp.bfloat16),
            pltpu.SemaphoreType.DMA((N_DEV,)),
            pltpu.SemaphoreType.DMA((N_DEV,)),
        ],
        compiler_params=pltpu.CompilerParams(collective_id=0),
    )(A, B)
